# baseline (speedup 1.0000x reference)
import functools

import jax
import jax.numpy as jnp
from jax import lax
from jax.experimental import pallas as pl
from jax.experimental.pallas import tpu as pltpu, tpu_sc as plsc

N = 10000
E = 160000
D = 256
DH = D // 2
NSC = 16
EPT = E // NSC
K = 80
CH = EPT // K
RPT = 624
RREM = N - NSC * RPT


def _sc_scatter_add(msgs, tgt3, zeros):
    mesh = plsc.VectorSubcoreMesh(core_axis_name="c", subcore_axis_name="s")

    @functools.partial(
        pl.kernel,
        mesh=mesh,
        out_type=jax.ShapeDtypeStruct((2, N, DH), jnp.float32),
        scratch_types=[
            pltpu.VMEM((CH, K), jnp.int32),
            pltpu.VMEM((2, K, DH), jnp.float32),
            pltpu.VMEM_SHARED((N, DH), jnp.float32),
            pltpu.SemaphoreType.DMA,
            pltpu.SemaphoreType.DMA,
        ],
    )
    def k(msgs_hbm, tgt_hbm, zeros_hbm, out_hbm, idx_v, bufs_v, acc_sh,
          gsem0, gsem1):
        c = lax.axis_index("c")
        s = lax.axis_index("s")
        col0 = pl.multiple_of(c * DH, DH)
        r0 = s * RPT
        ebase0 = s * EPT
        gsems = (gsem0, gsem1)

        def gather(j, b):
            eb = pl.multiple_of(ebase0 + j * K, 8)
            return pltpu.make_async_copy(
                msgs_hbm.at[pl.ds(eb, K), pl.ds(col0, DH)],
                bufs_v.at[b], gsems[b])

        with jax.named_scope("sc_init"):
            gather(0, 0).start()
            pltpu.sync_copy(zeros_hbm.at[pl.ds(r0, RPT), :],
                            acc_sh.at[pl.ds(r0, RPT), :])

            @pl.when(s == 0)
            def _init_rem():
                pltpu.sync_copy(zeros_hbm.at[pl.ds(NSC * RPT, RREM), :],
                                acc_sh.at[pl.ds(NSC * RPT, RREM), :])

            pltpu.sync_copy(tgt_hbm.at[s], idx_v)
            plsc.subcore_barrier()

        with jax.named_scope("sc_scatter_loop"):

            def pair_body(p, carry):
                for b in (0, 1):
                    j = 2 * p + b

                    @pl.when(j < CH)
                    def _do_chunk():
                        gather(j, b).wait()

                        @pl.when(j + 1 < CH)
                        def _prefetch():
                            gather(j + 1, 1 - b).start()

                        pltpu.sync_copy(bufs_v.at[b],
                                        acc_sh.at[idx_v.at[j]], add=True)
                return carry

            lax.fori_loop(0, (CH + 1) // 2, pair_body, 0)
            plsc.subcore_barrier()

        with jax.named_scope("sc_drain"):
            pltpu.sync_copy(acc_sh.at[pl.ds(r0, RPT), :],
                            out_hbm.at[c, pl.ds(r0, RPT), :])

            @pl.when(s == 0)
            def _drain_rem():
                pltpu.sync_copy(acc_sh.at[pl.ds(NSC * RPT, RREM), :],
                                out_hbm.at[c, pl.ds(NSC * RPT, RREM), :])

    return k(msgs, tgt3, zeros)


def _mh_matmul(h, U, b):
    NB = 1000

    def body(h_ref, u_ref, b_ref, o_ref):
        o_ref[...] = jnp.dot(h_ref[...], u_ref[...],
                             preferred_element_type=jnp.float32) + b_ref[1:2, :]

    return pl.pallas_call(
        body,
        grid=(N // NB,),
        in_specs=[
            pl.BlockSpec((NB, D), lambda i: (i, 0)),
            pl.BlockSpec((D, 3 * D), lambda i: (0, 0)),
            pl.BlockSpec((2, 3 * D), lambda i: (0, 0)),
        ],
        out_specs=pl.BlockSpec((NB, 3 * D), lambda i: (i, 0)),
        out_shape=jax.ShapeDtypeStruct((N, 3 * D), jnp.float32),
    )(h, U, b)


def _gru_combine(agg2, h, mh, W, b):
    NB = 1000

    def body(x_ref, h_ref, mh_ref, w_ref, b_ref, o_ref):
        x = jnp.concatenate([x_ref[0], x_ref[1]], axis=-1)
        hv = h_ref[...]
        mx = jnp.dot(x, w_ref[...], preferred_element_type=jnp.float32)
        mx = mx + b_ref[0:1, :]
        mh = mh_ref[...]
        z = jax.nn.sigmoid(mx[:, :D] + mh[:, :D])
        r = jax.nn.sigmoid(mx[:, D:2 * D] + mh[:, D:2 * D])
        hh = jnp.tanh(mx[:, 2 * D:] + r * mh[:, 2 * D:])
        o_ref[...] = z * hv + (1.0 - z) * hh

    return pl.pallas_call(
        body,
        grid=(N // NB,),
        in_specs=[
            pl.BlockSpec((2, NB, DH), lambda i: (0, i, 0)),
            pl.BlockSpec((NB, D), lambda i: (i, 0)),
            pl.BlockSpec((NB, 3 * D), lambda i: (i, 0)),
            pl.BlockSpec((D, 3 * D), lambda i: (0, 0)),
            pl.BlockSpec((2, 3 * D), lambda i: (0, 0)),
        ],
        out_specs=pl.BlockSpec((NB, D), lambda i: (i, 0)),
        out_shape=jax.ShapeDtypeStruct((N, D), jnp.float32),
    )(agg2, h, mh, W, b)


def kernel(atom_state, messages, connectivity, W, U, b):
    Bs, Ns, Ds = atom_state.shape
    Es = messages.shape[1]
    msgs = messages.reshape(Es, Ds)
    tgt = connectivity.reshape(Es, 2)[:, 1]
    tgt3 = tgt.reshape(NSC, CH, K)
    zeros = jnp.zeros((Ns, Ds // 2), jnp.float32)
    h = atom_state.reshape(Ns, Ds)
    agg2 = _sc_scatter_add(msgs, tgt3, zeros)
    mh = _mh_matmul(h, U, b)
    h_new = _gru_combine(agg2, h, mh, W, b)
    return h_new.reshape(Bs, Ns, Ds)


# speedup vs baseline: 1.0451x; 1.0451x over previous
import functools

import jax
import jax.numpy as jnp
from jax import lax
from jax.experimental import pallas as pl
from jax.experimental.pallas import tpu as pltpu, tpu_sc as plsc

N = 10000
E = 160000
D = 256
DH = D // 2
NSC = 16
EPT = E // NSC
K = 80
CH = EPT // K
RPT = 624
RREM = N - NSC * RPT


def _sc_scatter_add(msgs, tgt3, zeros):
    mesh = plsc.VectorSubcoreMesh(core_axis_name="c", subcore_axis_name="s")

    @functools.partial(
        pl.kernel,
        mesh=mesh,
        out_type=jax.ShapeDtypeStruct((2, N, DH), jnp.float32),
        scratch_types=[
            pltpu.VMEM((CH, K), jnp.int32),
            pltpu.VMEM((2, K, DH), jnp.float32),
            pltpu.VMEM_SHARED((N, DH), jnp.float32),
            pltpu.SemaphoreType.DMA,
            pltpu.SemaphoreType.DMA,
        ],
    )
    def k(msgs_hbm, tgt_hbm, zeros_hbm, out_hbm, idx_v, bufs_v, acc_sh,
          gsem0, gsem1):
        c = lax.axis_index("c")
        s = lax.axis_index("s")
        col0 = pl.multiple_of(c * DH, DH)
        r0 = s * RPT
        ebase0 = s * EPT
        gsems = (gsem0, gsem1)

        def gather(j, b):
            eb = pl.multiple_of(ebase0 + j * K, 8)
            return pltpu.make_async_copy(
                msgs_hbm.at[pl.ds(eb, K), pl.ds(col0, DH)],
                bufs_v.at[b], gsems[b])

        with jax.named_scope("sc_init"):
            gather(0, 0).start()
            pltpu.sync_copy(zeros_hbm.at[pl.ds(r0, RPT), :],
                            acc_sh.at[pl.ds(r0, RPT), :])

            @pl.when(s == 0)
            def _init_rem():
                pltpu.sync_copy(zeros_hbm.at[pl.ds(NSC * RPT, RREM), :],
                                acc_sh.at[pl.ds(NSC * RPT, RREM), :])

            pltpu.sync_copy(tgt_hbm.at[s], idx_v)
            plsc.subcore_barrier()

        with jax.named_scope("sc_scatter_loop"):

            def pair_body(p, carry):
                for b in (0, 1):
                    j = 2 * p + b

                    @pl.when(j < CH)
                    def _do_chunk():
                        gather(j, b).wait()

                        @pl.when(j + 1 < CH)
                        def _prefetch():
                            gather(j + 1, 1 - b).start()

                        pltpu.sync_copy(bufs_v.at[b],
                                        acc_sh.at[idx_v.at[j]], add=True)
                return carry

            lax.fori_loop(0, (CH + 1) // 2, pair_body, 0)
            plsc.subcore_barrier()

        with jax.named_scope("sc_drain"):
            pltpu.sync_copy(acc_sh.at[pl.ds(r0, RPT), :],
                            out_hbm.at[c, pl.ds(r0, RPT), :])

            @pl.when(s == 0)
            def _drain_rem():
                pltpu.sync_copy(acc_sh.at[pl.ds(NSC * RPT, RREM), :],
                                out_hbm.at[c, pl.ds(NSC * RPT, RREM), :])

    return k(msgs, tgt3, zeros)


def _gru_update(agg2, h, W, U, b):
    NB = 1000

    def body(x_ref, h_ref, w_ref, u_ref, b_ref, o_ref):
        x = jnp.concatenate([x_ref[0], x_ref[1]], axis=-1)
        hv = h_ref[...]
        mx = jnp.dot(x, w_ref[...], preferred_element_type=jnp.float32)
        mx = mx + b_ref[0:1, :]
        mh = jnp.dot(hv, u_ref[...], preferred_element_type=jnp.float32)
        mh = mh + b_ref[1:2, :]
        z = jax.nn.sigmoid(mx[:, :D] + mh[:, :D])
        r = jax.nn.sigmoid(mx[:, D:2 * D] + mh[:, D:2 * D])
        hh = jnp.tanh(mx[:, 2 * D:] + r * mh[:, 2 * D:])
        o_ref[...] = z * hv + (1.0 - z) * hh

    return pl.pallas_call(
        body,
        grid=(N // NB,),
        in_specs=[
            pl.BlockSpec((2, NB, DH), lambda i: (0, i, 0)),
            pl.BlockSpec((NB, D), lambda i: (i, 0)),
            pl.BlockSpec((D, 3 * D), lambda i: (0, 0)),
            pl.BlockSpec((D, 3 * D), lambda i: (0, 0)),
            pl.BlockSpec((2, 3 * D), lambda i: (0, 0)),
        ],
        out_specs=pl.BlockSpec((NB, D), lambda i: (i, 0)),
        out_shape=jax.ShapeDtypeStruct((N, D), jnp.float32),
    )(agg2, h, W, U, b)


def kernel(atom_state, messages, connectivity, W, U, b):
    Bs, Ns, Ds = atom_state.shape
    Es = messages.shape[1]
    msgs = messages.reshape(Es, Ds)
    tgt = connectivity.reshape(Es, 2)[:, 1]
    tgt3 = tgt.reshape(NSC, CH, K)
    zeros = jnp.zeros((Ns, Ds // 2), jnp.float32)
    agg2 = _sc_scatter_add(msgs, tgt3, zeros)
    h_new = _gru_update(agg2, atom_state.reshape(Ns, Ds), W, U, b)
    return h_new.reshape(Bs, Ns, Ds)


# speedup vs baseline: 1.0619x; 1.0161x over previous
import functools

import jax
import jax.numpy as jnp
from jax import lax
from jax.experimental import pallas as pl
from jax.experimental.pallas import tpu as pltpu, tpu_sc as plsc

N = 10000
E = 160000
D = 256
DH = D // 2
NSC = 16
EPT = E // NSC
K = 80
CH = EPT // K
RPT = 624
RREM = N - NSC * RPT


def _sc_scatter_add(msgs, tgt3, zeros):
    mesh = plsc.VectorSubcoreMesh(core_axis_name="c", subcore_axis_name="s")

    @functools.partial(
        pl.kernel,
        mesh=mesh,
        out_type=jax.ShapeDtypeStruct((2, N, DH), jnp.float32),
        scratch_types=[
            pltpu.VMEM((CH, K), jnp.int32),
            pltpu.VMEM((2, K, DH), jnp.float32),
            pltpu.VMEM_SHARED((N, DH), jnp.float32),
            pltpu.SemaphoreType.DMA,
            pltpu.SemaphoreType.DMA,
        ],
    )
    def k(msgs_hbm, tgt_hbm, zeros_hbm, out_hbm, idx_v, bufs_v, acc_sh,
          gsem0, gsem1):
        c = lax.axis_index("c")
        s = lax.axis_index("s")
        col0 = pl.multiple_of(c * DH, DH)
        r0 = s * RPT
        ebase0 = s * EPT
        gsems = (gsem0, gsem1)

        def gather(j, b):
            eb = pl.multiple_of(ebase0 + j * K, 8)
            return pltpu.make_async_copy(
                msgs_hbm.at[pl.ds(eb, K), pl.ds(col0, DH)],
                bufs_v.at[b], gsems[b])

        with jax.named_scope("sc_init"):
            gather(0, 0).start()
            pltpu.sync_copy(zeros_hbm.at[pl.ds(r0, RPT), :],
                            acc_sh.at[pl.ds(r0, RPT), :])

            @pl.when(s == 0)
            def _init_rem():
                pltpu.sync_copy(zeros_hbm.at[pl.ds(NSC * RPT, RREM), :],
                                acc_sh.at[pl.ds(NSC * RPT, RREM), :])

            pltpu.sync_copy(tgt_hbm.at[s], idx_v)
            plsc.subcore_barrier()

        with jax.named_scope("sc_scatter_loop"):

            def pair_body(p, carry):
                for b in (0, 1):
                    j = 2 * p + b

                    @pl.when(j < CH)
                    def _do_chunk():
                        gather(j, b).wait()

                        @pl.when(j + 1 < CH)
                        def _prefetch():
                            gather(j + 1, 1 - b).start()

                        pltpu.sync_copy(bufs_v.at[b],
                                        acc_sh.at[idx_v.at[j]], add=True)
                return carry

            lax.fori_loop(0, (CH + 1) // 2, pair_body, 0)
            plsc.subcore_barrier()

        with jax.named_scope("sc_drain"):
            pltpu.sync_copy(acc_sh.at[pl.ds(r0, RPT), :],
                            out_hbm.at[c, pl.ds(r0, RPT), :])

            @pl.when(s == 0)
            def _drain_rem():
                pltpu.sync_copy(acc_sh.at[pl.ds(NSC * RPT, RREM), :],
                                out_hbm.at[c, pl.ds(NSC * RPT, RREM), :])

    return k(msgs, tgt3, zeros)


def _gru_update(agg2, h, W, U, b):
    NB = 2000

    def body(x_ref, h_ref, w_ref, u_ref, b_ref, o_ref):
        x = jnp.concatenate([x_ref[0], x_ref[1]], axis=-1)
        hv = h_ref[...]
        mx = jnp.dot(x, w_ref[...], preferred_element_type=jnp.float32)
        mx = mx + b_ref[0:1, :]
        mh = jnp.dot(hv, u_ref[...], preferred_element_type=jnp.float32)
        mh = mh + b_ref[1:2, :]
        z = jax.nn.sigmoid(mx[:, :D] + mh[:, :D])
        r = jax.nn.sigmoid(mx[:, D:2 * D] + mh[:, D:2 * D])
        hh = jnp.tanh(mx[:, 2 * D:] + r * mh[:, 2 * D:])
        o_ref[...] = z * hv + (1.0 - z) * hh

    return pl.pallas_call(
        body,
        grid=(N // NB,),
        in_specs=[
            pl.BlockSpec((2, NB, DH), lambda i: (0, i, 0)),
            pl.BlockSpec((NB, D), lambda i: (i, 0)),
            pl.BlockSpec((D, 3 * D), lambda i: (0, 0)),
            pl.BlockSpec((D, 3 * D), lambda i: (0, 0)),
            pl.BlockSpec((2, 3 * D), lambda i: (0, 0)),
        ],
        out_specs=pl.BlockSpec((NB, D), lambda i: (i, 0)),
        out_shape=jax.ShapeDtypeStruct((N, D), jnp.float32),
    )(agg2, h, W, U, b)


def kernel(atom_state, messages, connectivity, W, U, b):
    Bs, Ns, Ds = atom_state.shape
    Es = messages.shape[1]
    msgs = messages.reshape(Es, Ds)
    tgt = connectivity.reshape(Es, 2)[:, 1]
    tgt3 = tgt.reshape(NSC, CH, K)
    zeros = jnp.zeros((Ns, Ds // 2), jnp.float32)
    agg2 = _sc_scatter_add(msgs, tgt3, zeros)
    h_new = _gru_update(agg2, atom_state.reshape(Ns, Ds), W, U, b)
    return h_new.reshape(Bs, Ns, Ds)
